# Initial kernel scaffold; baseline (speedup 1.0000x reference)
import jax
import jax.numpy as jnp
from jax.experimental import pallas as pl


def kernel(rel, arg1, fact_rel, fact_arg1, fact_arg2):
    raise NotImplementedError("write your pallas kernel here")


# speedup vs baseline: 1.6493x; 1.6493x over previous
import functools

import jax
import jax.numpy as jnp
from jax import lax
from jax.experimental import pallas as pl
from jax.experimental.pallas import tpu as pltpu
from jax.experimental.pallas import tpu_sc as plsc

K_NEIGH = 5
SLOPE = 1.0

_BT = 256
_FB = 2048
_PAD_K = 8

_F32_INF = float("inf")
_I32_BIG = 2**30


def _select_smallest(vals, idxs, pos, n_out):
    out_v, out_i = [], []
    for _ in range(n_out):
        m = jnp.min(vals, axis=1, keepdims=True)
        hit = vals == m
        ai = jnp.min(jnp.where(hit, idxs, _I32_BIG), axis=1, keepdims=True)
        ap = jnp.min(jnp.where(hit & (idxs == ai), pos, _I32_BIG), axis=1,
                     keepdims=True)
        out_v.append(m)
        out_i.append(ai)
        vals = jnp.where(pos == ap, _F32_INF, vals)
    return out_v, out_i


def _topk_body(n_fb, f_real, q_ref, k_ref, scores_ref, idx_ref,
               rv_ref, ri_ref):
    j = pl.program_id(1)

    @pl.when(j == 0)
    def _init():
        rv_ref[...] = jnp.full((_BT, _PAD_K), _F32_INF, jnp.float32)
        ri_ref[...] = jnp.full((_BT, _PAD_K), _I32_BIG, jnp.int32)

    q = q_ref[...]
    k = k_ref[...]
    q2 = jnp.sum(q * q, axis=1, keepdims=True)
    k2 = jnp.sum(k * k, axis=1)
    qk = lax.dot_general(q, k, (((1,), (1,)), ((), ())),
                         preferred_element_type=jnp.float32)
    d2 = (q2 - 2.0 * qk) + k2[None, :]

    col = j * _FB + lax.broadcasted_iota(jnp.int32, (_BT, _FB), 1)
    d2 = jnp.where(col < f_real, d2, _F32_INF)

    work = d2
    bv, bi = [], []
    for _ in range(K_NEIGH):
        m = jnp.min(work, axis=1, keepdims=True)
        ai = jnp.min(jnp.where(work == m, col, _I32_BIG), axis=1,
                     keepdims=True)
        bv.append(m)
        bi.append(ai)
        work = jnp.where(col == ai, _F32_INF, work)

    cand_v = jnp.concatenate([rv_ref[...]] + bv, axis=1)
    cand_i = jnp.concatenate([ri_ref[...]] + bi, axis=1)
    pos = lax.broadcasted_iota(jnp.int32, (_BT, _PAD_K + K_NEIGH), 1)
    new_v, new_i = _select_smallest(cand_v, cand_i, pos, K_NEIGH)

    pad_v = jnp.full((_BT, _PAD_K - K_NEIGH), _F32_INF, jnp.float32)
    pad_i = jnp.full((_BT, _PAD_K - K_NEIGH), _I32_BIG, jnp.int32)
    merged_v = jnp.concatenate(new_v + [pad_v], axis=1)
    merged_i = jnp.concatenate(new_i + [pad_i], axis=1)
    rv_ref[...] = merged_v
    ri_ref[...] = merged_i

    @pl.when(j == n_fb - 1)
    def _emit():
        scores_ref[...] = jnp.exp(merged_v * jnp.float32(-0.5 / SLOPE**2))
        idx_ref[...] = merged_i


def _topk_scores(q, keys, f_real):
    b, dq = q.shape
    fp = keys.shape[0]
    n_fb = fp // _FB
    grid = (b // _BT, n_fb)
    return pl.pallas_call(
        functools.partial(_topk_body, n_fb, f_real),
        grid=grid,
        in_specs=[
            pl.BlockSpec((_BT, dq), lambda i, j: (i, 0)),
            pl.BlockSpec((_FB, dq), lambda i, j: (j, 0)),
        ],
        out_specs=[
            pl.BlockSpec((_BT, _PAD_K), lambda i, j: (i, 0)),
            pl.BlockSpec((_BT, _PAD_K), lambda i, j: (i, 0)),
        ],
        out_shape=[
            jax.ShapeDtypeStruct((b, _PAD_K), jnp.float32),
            jax.ShapeDtypeStruct((b, _PAD_K), jnp.int32),
        ],
        scratch_shapes=[
            pltpu.VMEM((_BT, _PAD_K), jnp.float32),
            pltpu.VMEM((_BT, _PAD_K), jnp.int32),
        ],
    )(q, keys)


def _sc_gather(flat_idx, table):
    n, = flat_idx.shape
    d = table.shape[1]
    info = plsc.get_sparse_core_info()
    nw = info.num_cores * info.num_subcores
    rows = n // nw
    mesh = plsc.VectorSubcoreMesh(core_axis_name="c", subcore_axis_name="s")

    @functools.partial(
        pl.kernel,
        out_type=jax.ShapeDtypeStruct((n, d), jnp.float32),
        mesh=mesh,
        compiler_params=pltpu.CompilerParams(use_tc_tiling_on_sc=False),
        scratch_types=[
            pltpu.VMEM((rows,), jnp.int32),
            pltpu.VMEM((rows, d), jnp.float32),
            pltpu.SemaphoreType.DMA,
        ],
    )
    def gather_kernel(idx_hbm, table_hbm, out_hbm, idx_v, rows_v, sem):
        wid = lax.axis_index("s") * info.num_cores + lax.axis_index("c")
        base = wid * rows
        pltpu.sync_copy(idx_hbm.at[pl.ds(base, rows)], idx_v)
        pltpu.async_copy(table_hbm.at[idx_v], rows_v, sem).wait()
        pltpu.sync_copy(rows_v, out_hbm.at[pl.ds(base, rows)])

    return gather_kernel(flat_idx, table)


def kernel(rel, arg1, fact_rel, fact_arg1, fact_arg2):
    b, d = rel.shape
    f = fact_rel.shape[0]

    q = jnp.concatenate([rel, arg1], axis=1)
    keys = jnp.concatenate([fact_rel, fact_arg1], axis=1)
    fp = ((f + _FB - 1) // _FB) * _FB
    keys = jnp.pad(keys, ((0, fp - f), (0, 0)))

    scores8, idx8 = _topk_scores(q, keys, f)
    scores = scores8[:, :K_NEIGH]
    flat_idx = idx8[:, :K_NEIGH].reshape(-1)

    subs = _sc_gather(flat_idx, fact_arg2).reshape(b, K_NEIGH, d)
    return scores, subs
